# baseline (speedup 1.0000x reference)
import functools

import jax
import jax.numpy as jnp
from jax import lax
from jax.experimental import pallas as pl
from jax.experimental.pallas import tpu as pltpu
from jax.experimental.pallas import tpu_sc as plsc

_EPS = 1e-5
_F32 = jnp.float32


def _dot_x(a, b):
    return jnp.dot(a, b, preferred_element_type=_F32,
                   precision=jax.lax.Precision.HIGHEST)


def _fold(a, width):
    g = a.shape[-1] // width
    out = a[..., :width]
    for k in range(1, g):
        out = out + a[..., k * width:(k + 1) * width]
    return out


def _tile_lanes(a, reps):
    return jnp.concatenate([a] * reps, axis=-1)


def _bn_scale_shift(s, gamma, beta, count):
    mean = s[0:1, :] / count
    var = s[1:2, :] / count - mean * mean
    inv = lax.rsqrt(var + _EPS)
    scale = gamma * inv
    return scale, beta - mean * scale


def _sc_gather(xk, xv, q16, edges2d):
    n, c = xk.shape
    qw = q16.shape[1]
    ns = edges2d.shape[1]
    b = n * ns
    nw = 32
    per_w = b // nw
    ch = 1000
    assert per_w % ch == 0 and ch % 8 == 0 and per_w * nw == b
    nch = per_w // ch
    mesh = plsc.VectorSubcoreMesh(core_axis_name="c", subcore_axis_name="s")

    out_type = (
        jax.ShapeDtypeStruct((b, c), _F32),
        jax.ShapeDtypeStruct((b, c), _F32),
        jax.ShapeDtypeStruct((b, qw), _F32),
    )
    scratch = [
        pltpu.VMEM((ch,), jnp.int32),
        pltpu.VMEM((ch, c), _F32),
        pltpu.VMEM((ch, c), _F32),
        pltpu.VMEM((ch, qw), _F32),
        pltpu.SemaphoreType.DMA,
        pltpu.SemaphoreType.DMA,
        pltpu.SemaphoreType.DMA,
        pltpu.SemaphoreType.DMA,
        pltpu.SemaphoreType.DMA,
        pltpu.SemaphoreType.DMA,
    ]

    def body(xk_h, xv_h, q16_h, idx_h, gk_h, gv_h, gq_h,
             idx_v, bk, bv, bq, s1, s2, s3, w1, w2, w3):
        wid = lax.axis_index("s") * 2 + lax.axis_index("c")
        base = wid * per_w

        pltpu.async_copy(bk, gk_h.at[pl.ds(base, ch)], w1)
        pltpu.async_copy(bv, gv_h.at[pl.ds(base, ch)], w2)
        pltpu.async_copy(bq, gq_h.at[pl.ds(base, ch)], w3)

        def step(k, carry):
            off = base + k * ch
            pltpu.make_async_copy(bk, gk_h.at[pl.ds(off, ch)], w1).wait()
            pltpu.make_async_copy(bv, gv_h.at[pl.ds(off, ch)], w2).wait()
            pltpu.make_async_copy(bq, gq_h.at[pl.ds(off, ch)], w3).wait()
            pltpu.sync_copy(idx_h.at[pl.ds(off, ch)], idx_v)
            c1 = pltpu.async_copy(xk_h.at[idx_v], bk, s1)
            c2 = pltpu.async_copy(xv_h.at[idx_v], bv, s2)
            c3 = pltpu.async_copy(q16_h.at[idx_v], bq, s3)
            c1.wait()
            c2.wait()
            c3.wait()
            pltpu.async_copy(bk, gk_h.at[pl.ds(off, ch)], w1)
            pltpu.async_copy(bv, gv_h.at[pl.ds(off, ch)], w2)
            pltpu.async_copy(bq, gq_h.at[pl.ds(off, ch)], w3)
            return carry

        lax.fori_loop(0, nch, step, 0)
        last = base + (nch - 1) * ch
        pltpu.make_async_copy(bk, gk_h.at[pl.ds(last, ch)], w1).wait()
        pltpu.make_async_copy(bv, gv_h.at[pl.ds(last, ch)], w2).wait()
        pltpu.make_async_copy(bq, gq_h.at[pl.ds(last, ch)], w3).wait()

    fn = pl.kernel(
        body, out_type=out_type, mesh=mesh, scratch_types=scratch,
        compiler_params=pltpu.CompilerParams(use_tc_tiling_on_sc=False))
    return fn(xk, xv, q16, edges2d.reshape(-1))


_gather_rows = _sc_gather


def _tc_qkv(x, p4, wq, bq, wk, bk, wv, bv, wp1_16, bn):
    n, c = x.shape
    qw = wp1_16.shape[1]
    grid = (n // bn,)
    blk = lambda i: (i, 0)
    zero = lambda i: (0, 0)

    def body(x_r, p4_r, wq_r, bq_r, wk_r, bk_r, wv_r, bv_r, wp_r,
             xq_r, xk_r, xv_r, q_r):
        xb = x_r[...]
        xq_r[...] = jnp.dot(xb, wq_r[...], preferred_element_type=_F32) + bq_r[...]
        xk_r[...] = jnp.dot(xb, wk_r[...], preferred_element_type=_F32) + bk_r[...]
        xv_r[...] = jnp.dot(xb, wv_r[...], preferred_element_type=_F32) + bv_r[...]
        q_r[...] = jnp.dot(p4_r[...], wp_r[...], preferred_element_type=_F32)

    return pl.pallas_call(
        body,
        grid=grid,
        in_specs=[
            pl.BlockSpec((bn, c), blk),
            pl.BlockSpec((bn, 4), blk),
            pl.BlockSpec((c, c), zero),
            pl.BlockSpec((1, c), zero),
            pl.BlockSpec((c, c), zero),
            pl.BlockSpec((1, c), zero),
            pl.BlockSpec((c, c), zero),
            pl.BlockSpec((1, c), zero),
            pl.BlockSpec((4, qw), zero),
        ],
        out_specs=[
            pl.BlockSpec((bn, c), blk),
            pl.BlockSpec((bn, c), blk),
            pl.BlockSpec((bn, c), blk),
            pl.BlockSpec((bn, qw), blk),
        ],
        out_shape=[
            jax.ShapeDtypeStruct((n, c), _F32),
            jax.ShapeDtypeStruct((n, c), _F32),
            jax.ShapeDtypeStruct((n, c), _F32),
            jax.ShapeDtypeStruct((n, qw), _F32),
        ],
    )(x, p4, wq, bq, wk, bk, wv, bv, wp1_16)


def _tc_stats1(gq_wide, q16, msel, msel2, bp1_64, ns, bn):
    n, wide = gq_wide.shape
    qw = wide // ns
    grid = (n // bn,)
    blk = lambda i: (i, 0)
    zero = lambda i: (0, 0)

    def body(gq_r, q16_r, msel_r, msel2_r, b_r, pr3_r, s_r):
        pr3 = (_dot_x(gq_r[...], msel_r[...])
               - _dot_x(q16_r[...], msel2_r[...]) + b_r[...])
        pr3_r[...] = pr3

        @pl.when(pl.program_id(0) == 0)
        def _():
            s_r[...] = jnp.zeros_like(s_r)

        s0 = jnp.sum(pr3, axis=0, keepdims=True)
        s1 = jnp.sum(pr3 * pr3, axis=0, keepdims=True)
        s_r[...] += jnp.concatenate([s0, s1], axis=0)

    return pl.pallas_call(
        body,
        grid=grid,
        in_specs=[
            pl.BlockSpec((bn, wide), blk),
            pl.BlockSpec((bn, qw), blk),
            pl.BlockSpec((wide, 4 * ns), zero),
            pl.BlockSpec((qw, 4 * ns), zero),
            pl.BlockSpec((1, 4 * ns), zero),
        ],
        out_specs=[
            pl.BlockSpec((bn, 4 * ns), blk),
            pl.BlockSpec((2, 4 * ns), zero),
        ],
        out_shape=[
            jax.ShapeDtypeStruct((n, 4 * ns), _F32),
            jax.ShapeDtypeStruct((2, 4 * ns), _F32),
        ],
    )(gq_wide, q16, msel, msel2, bp1_64)


def _tc_stats2(pr3, xq, gk, s_a, pk, tx, bp2_t, gp4, bp4, ns, rtot, bn):
    n, wide = gk.shape
    c = xq.shape[1]
    grid = (n // bn,)
    blk = lambda i: (i, 0)
    zero = lambda i: (0, 0)

    def body(pr3_r, xq_r, gk_r, sa_r, pk_r, tx_r, b2_r, gp_r, bp_r, s_r):
        sc4, sh4 = _bn_scale_shift(_fold(sa_r[...], 4), gp_r[...], bp_r[...], rtot)
        r = jnp.maximum(pr3_r[...] * _tile_lanes(sc4, ns) + _tile_lanes(sh4, ns), 0.0)
        pr = jnp.dot(r, pk_r[...], preferred_element_type=_F32) + b2_r[...]
        w = (gk_r[...].astype(_F32)
             - _dot_x(xq_r[...], tx_r[...]) + pr)

        @pl.when(pl.program_id(0) == 0)
        def _():
            s_r[...] = jnp.zeros_like(s_r)

        s0 = _fold(jnp.sum(w, axis=0, keepdims=True), c)
        s1 = _fold(jnp.sum(w * w, axis=0, keepdims=True), c)
        s_r[...] += jnp.concatenate([s0, s1], axis=0)

    return pl.pallas_call(
        body,
        grid=grid,
        in_specs=[
            pl.BlockSpec((bn, 4 * ns), blk),
            pl.BlockSpec((bn, c), blk),
            pl.BlockSpec((bn, wide), blk),
            pl.BlockSpec((2, 4 * ns), zero),
            pl.BlockSpec(pk.shape, zero),
            pl.BlockSpec(tx.shape, zero),
            pl.BlockSpec((1, wide), zero),
            pl.BlockSpec((1, 4), zero),
            pl.BlockSpec((1, 4), zero),
        ],
        out_specs=pl.BlockSpec((2, c), zero),
        out_shape=jax.ShapeDtypeStruct((2, c), _F32),
    )(pr3, xq, gk, s_a, pk, tx, bp2_t, gp4, bp4)


def _tc_wchain(pr3, xq, gk, s_a, s_b, pk, tx, k1, bp2_t, gp4, bp4,
               g1r, b1r, bw1_t, ns, m, rtot, bn):
    n, wide = gk.shape
    c = xq.shape[1]
    grid = (n // bn,)
    blk = lambda i: (i, 0)
    zero = lambda i: (0, 0)

    def body(pr3_r, xq_r, gk_r, sa_r, sb_r, pk_r, tx_r, k1_r, b2_r,
             gp_r, bp_r, g1_r2, b1_r2, bw1_r, w4a_r, s_r):
        sc4, sh4 = _bn_scale_shift(_fold(sa_r[...], 4), gp_r[...], bp_r[...], rtot)
        r = jnp.maximum(pr3_r[...] * _tile_lanes(sc4, ns) + _tile_lanes(sh4, ns), 0.0)
        pr = jnp.dot(r, pk_r[...], preferred_element_type=_F32) + b2_r[...]
        w = (gk_r[...].astype(_F32)
             - _dot_x(xq_r[...], tx_r[...]) + pr)
        sc32, sh32 = _bn_scale_shift(sb_r[...], g1_r2[...], b1_r2[...], rtot)
        r2 = jnp.maximum(w * _tile_lanes(sc32, ns) + _tile_lanes(sh32, ns), 0.0)
        w4a = jnp.dot(r2, k1_r[...], preferred_element_type=_F32) + bw1_r[...]
        w4a_r[...] = w4a

        @pl.when(pl.program_id(0) == 0)
        def _():
            s_r[...] = jnp.zeros_like(s_r)

        s0 = _fold(jnp.sum(w4a, axis=0, keepdims=True), m)
        s1 = _fold(jnp.sum(w4a * w4a, axis=0, keepdims=True), m)
        s_r[...] += jnp.concatenate([s0, s1], axis=0)

    return pl.pallas_call(
        body,
        grid=grid,
        in_specs=[
            pl.BlockSpec((bn, 4 * ns), blk),
            pl.BlockSpec((bn, c), blk),
            pl.BlockSpec((bn, wide), blk),
            pl.BlockSpec((2, 4 * ns), zero),
            pl.BlockSpec((2, c), zero),
            pl.BlockSpec(pk.shape, zero),
            pl.BlockSpec(tx.shape, zero),
            pl.BlockSpec(k1.shape, zero),
            pl.BlockSpec((1, wide), zero),
            pl.BlockSpec((1, 4), zero),
            pl.BlockSpec((1, 4), zero),
            pl.BlockSpec((1, c), zero),
            pl.BlockSpec((1, c), zero),
            pl.BlockSpec((1, m * ns), zero),
        ],
        out_specs=[
            pl.BlockSpec((bn, m * ns), blk),
            pl.BlockSpec((2, m), zero),
        ],
        out_shape=[
            jax.ShapeDtypeStruct((n, m * ns), _F32),
            jax.ShapeDtypeStruct((2, m), _F32),
        ],
    )(pr3, xq, gk, s_a, s_b, pk, tx, k1, bp2_t, gp4, bp4, g1r, b1r, bw1_t)


def _tc_final(w4a, pr3, gv, s_a, s_c, pk, k2, rw, d64, fold32, bp2_t,
              gp4, bp4, g2r, b2r, bw2_t, ns, m, rtot, bn):
    n, wide = gv.shape
    c = wide // ns
    grid = (n // bn,)
    blk = lambda i: (i, 0)
    zero = lambda i: (0, 0)

    def body(w4a_r, pr3_r, gv_r, sa_r, sc_r, pk_r, k2_r, rw_r, d64_r,
             f32_r, b2_r, gp_r, bp_r, g2_r2, b2_r2, bw2_r, out_r):
        sc4, sh4 = _bn_scale_shift(sc_r[...], g2_r2[...], b2_r2[...], rtot)
        v = jnp.maximum(w4a_r[...] * _tile_lanes(sc4, ns) + _tile_lanes(sh4, ns), 0.0)
        w64 = jnp.dot(v, k2_r[...], preferred_element_type=_F32) + bw2_r[...]
        e = jnp.exp(w64)
        den = _dot_x(e, d64_r[...])
        sm = e / den
        w512 = jnp.dot(sm, rw_r[...], preferred_element_type=_F32)
        sca, sha = _bn_scale_shift(_fold(sa_r[...], 4), gp_r[...], bp_r[...], rtot)
        r = jnp.maximum(pr3_r[...] * _tile_lanes(sca, ns) + _tile_lanes(sha, ns), 0.0)
        pr = jnp.dot(r, pk_r[...], preferred_element_type=_F32) + b2_r[...]
        t = gv_r[...].astype(_F32) + pr
        out_r[...] = _dot_x(t * w512, f32_r[...])

    return pl.pallas_call(
        body,
        grid=grid,
        in_specs=[
            pl.BlockSpec((bn, m * ns), blk),
            pl.BlockSpec((bn, 4 * ns), blk),
            pl.BlockSpec((bn, wide), blk),
            pl.BlockSpec((2, 4 * ns), zero),
            pl.BlockSpec((2, m), zero),
            pl.BlockSpec(pk.shape, zero),
            pl.BlockSpec(k2.shape, zero),
            pl.BlockSpec(rw.shape, zero),
            pl.BlockSpec(d64.shape, zero),
            pl.BlockSpec(fold32.shape, zero),
            pl.BlockSpec((1, wide), zero),
            pl.BlockSpec((1, 4), zero),
            pl.BlockSpec((1, 4), zero),
            pl.BlockSpec((1, m), zero),
            pl.BlockSpec((1, m), zero),
            pl.BlockSpec((1, m * ns), zero),
        ],
        out_specs=pl.BlockSpec((bn, c), blk),
        out_shape=jax.ShapeDtypeStruct((n, c), _F32),
    )(w4a, pr3, gv, s_a, s_c, pk, k2, rw, d64, fold32, bp2_t, gp4, bp4,
      g2r, b2r, bw2_t)


def kernel(p, x, edges, Wq, bq, Wk, bk, Wv, bv, Wp1, bp1, gp, bp, Wp2, bp2,
           g1, b1, Ww1, bw1, g2, b2, Ww2, bw2):
    n, c = x.shape
    ns = edges.shape[1]
    m = Ww1.shape[1]
    s = c // m
    rtot = float(n * ns)
    bn = 4000
    assert n % bn == 0

    qw = 16
    eye_ns = jnp.eye(ns, dtype=_F32)
    p4 = jnp.pad(p, ((0, 0), (0, 1)))
    wp1_16 = jnp.pad(Wp1, ((0, 1), (0, qw - 3)))
    wp2_4 = jnp.pad(Wp2, ((0, 1), (0, 0)))
    bp1_64 = jnp.tile(jnp.pad(bp1, (0, 1)), ns)[None]
    bp2_t = jnp.tile(bp2, ns)[None]
    bw1_t = jnp.tile(bw1, ns)[None]
    bw2_t = jnp.tile(bw2, ns)[None]
    gp4 = jnp.pad(gp, (0, 1))[None]
    bp4 = jnp.pad(bp, (0, 1))[None]
    pk = jnp.kron(eye_ns, wp2_4)
    k1 = jnp.kron(eye_ns, Ww1)
    k2 = jnp.kron(eye_ns, Ww2)
    rw = jnp.kron(eye_ns, jnp.kron(jnp.ones((1, s), _F32),
                                   jnp.eye(m, dtype=_F32)))
    msel = jnp.kron(eye_ns, jnp.eye(qw, 4, dtype=_F32))
    msel2 = jnp.kron(jnp.ones((1, ns), _F32),
                     jnp.eye(qw, 4, dtype=_F32))
    tx = jnp.kron(jnp.ones((1, ns), _F32), jnp.eye(c, dtype=_F32))
    d64 = jnp.kron(jnp.ones((ns, ns), _F32), jnp.eye(m, dtype=_F32))
    fold32 = jnp.kron(jnp.ones((ns, 1), _F32), jnp.eye(c, dtype=_F32))

    xq, xk, xv, q16 = _tc_qkv(x, p4, Wq, bq[None], Wk, bk[None], Wv, bv[None],
                              wp1_16, bn)
    gk, gv, gq = _gather_rows(xk, xv, q16, edges.astype(jnp.int32))
    gk_w = gk.reshape(n, ns * c)
    gv_w = gv.reshape(n, ns * c)
    gq_w = gq.reshape(n, ns * qw)

    pr3, s_a = _tc_stats1(gq_w, q16, msel, msel2, bp1_64, ns, bn)
    s_b = _tc_stats2(pr3, xq, gk_w, s_a, pk, tx, bp2_t, gp4, bp4, ns, rtot, bn)
    w4a, s_c = _tc_wchain(pr3, xq, gk_w, s_a, s_b, pk, tx, k1, bp2_t,
                          gp4, bp4, g1[None], b1[None], bw1_t, ns, m, rtot, bn)
    out = _tc_final(w4a, pr3, gv_w, s_a, s_c, pk, k2, rw, d64, fold32, bp2_t,
                    gp4, bp4, g2[None], b2[None], bw2_t, ns, m, rtot, bn)
    return out


# speedup vs baseline: 1.1318x; 1.1318x over previous
import functools

import jax
import jax.numpy as jnp
from jax import lax
from jax.experimental import pallas as pl
from jax.experimental.pallas import tpu as pltpu
from jax.experimental.pallas import tpu_sc as plsc

_EPS = 1e-5
_F32 = jnp.float32


def _dot_x(a, b):
    return jnp.dot(a, b, preferred_element_type=_F32,
                   precision=jax.lax.Precision.HIGHEST)


def _fold(a, width):
    g = a.shape[-1] // width
    out = a[..., :width]
    for k in range(1, g):
        out = out + a[..., k * width:(k + 1) * width]
    return out


def _tile_lanes(a, reps):
    return jnp.concatenate([a] * reps, axis=-1)


def _bn_scale_shift(s, gamma, beta, count):
    mean = s[0:1, :] / count
    var = s[1:2, :] / count - mean * mean
    inv = lax.rsqrt(var + _EPS)
    scale = gamma * inv
    return scale, beta - mean * scale


def _sc_gather(xk, xv, q16, edges2d):
    n, c = xk.shape
    qw = q16.shape[1]
    ns = edges2d.shape[1]
    b = n * ns
    nw = 32
    per_w = b // nw
    ch = 1000
    assert per_w % ch == 0 and ch % 8 == 0 and per_w * nw == b
    nch = per_w // ch
    mesh = plsc.VectorSubcoreMesh(core_axis_name="c", subcore_axis_name="s")

    out_type = (
        jax.ShapeDtypeStruct((b, c), _F32),
        jax.ShapeDtypeStruct((b, c), _F32),
        jax.ShapeDtypeStruct((b, qw), _F32),
    )
    scratch = [
        pltpu.VMEM((ch,), jnp.int32),
        pltpu.VMEM((ch, c), _F32),
        pltpu.VMEM((ch, c), _F32),
        pltpu.VMEM((ch, qw), _F32),
        pltpu.SemaphoreType.DMA,
        pltpu.SemaphoreType.DMA,
        pltpu.SemaphoreType.DMA,
        pltpu.SemaphoreType.DMA,
        pltpu.SemaphoreType.DMA,
        pltpu.SemaphoreType.DMA,
    ]

    def body(xk_h, xv_h, q16_h, idx_h, gk_h, gv_h, gq_h,
             idx_v, bk, bv, bq, s1, s2, s3, w1, w2, w3):
        wid = lax.axis_index("s") * 2 + lax.axis_index("c")
        base = wid * per_w

        pltpu.async_copy(bk, gk_h.at[pl.ds(base, ch)], w1)
        pltpu.async_copy(bv, gv_h.at[pl.ds(base, ch)], w2)
        pltpu.async_copy(bq, gq_h.at[pl.ds(base, ch)], w3)

        def step(k, carry):
            off = base + k * ch
            pltpu.make_async_copy(bk, gk_h.at[pl.ds(off, ch)], w1).wait()
            pltpu.make_async_copy(bv, gv_h.at[pl.ds(off, ch)], w2).wait()
            pltpu.make_async_copy(bq, gq_h.at[pl.ds(off, ch)], w3).wait()
            pltpu.sync_copy(idx_h.at[pl.ds(off, ch)], idx_v)
            c1 = pltpu.async_copy(xk_h.at[idx_v], bk, s1)
            c2 = pltpu.async_copy(xv_h.at[idx_v], bv, s2)
            c3 = pltpu.async_copy(q16_h.at[idx_v], bq, s3)
            c1.wait()
            c2.wait()
            c3.wait()
            pltpu.async_copy(bk, gk_h.at[pl.ds(off, ch)], w1)
            pltpu.async_copy(bv, gv_h.at[pl.ds(off, ch)], w2)
            pltpu.async_copy(bq, gq_h.at[pl.ds(off, ch)], w3)
            return carry

        lax.fori_loop(0, nch, step, 0)
        last = base + (nch - 1) * ch
        pltpu.make_async_copy(bk, gk_h.at[pl.ds(last, ch)], w1).wait()
        pltpu.make_async_copy(bv, gv_h.at[pl.ds(last, ch)], w2).wait()
        pltpu.make_async_copy(bq, gq_h.at[pl.ds(last, ch)], w3).wait()

    fn = pl.kernel(
        body, out_type=out_type, mesh=mesh, scratch_types=scratch,
        compiler_params=pltpu.CompilerParams(use_tc_tiling_on_sc=False))
    return fn(xk, xv, q16, edges2d.reshape(-1))


_gather_rows = _sc_gather


def _tc_qkv(x, p4, wq, bq, wk, bk, wv, bv, wp1_16, bn):
    n, c = x.shape
    qw = wp1_16.shape[1]
    grid = (n // bn,)
    blk = lambda i: (i, 0)
    zero = lambda i: (0, 0)

    def body(x_r, p4_r, wq_r, bq_r, wk_r, bk_r, wv_r, bv_r, wp_r,
             xq_r, xk_r, xv_r, q_r):
        xb = x_r[...]
        xq_r[...] = jnp.dot(xb, wq_r[...], preferred_element_type=_F32) + bq_r[...]
        xk_r[...] = jnp.dot(xb, wk_r[...], preferred_element_type=_F32) + bk_r[...]
        xv_r[...] = jnp.dot(xb, wv_r[...], preferred_element_type=_F32) + bv_r[...]
        q_r[...] = jnp.dot(p4_r[...], wp_r[...], preferred_element_type=_F32)

    return pl.pallas_call(
        body,
        grid=grid,
        in_specs=[
            pl.BlockSpec((bn, c), blk),
            pl.BlockSpec((bn, 4), blk),
            pl.BlockSpec((c, c), zero),
            pl.BlockSpec((1, c), zero),
            pl.BlockSpec((c, c), zero),
            pl.BlockSpec((1, c), zero),
            pl.BlockSpec((c, c), zero),
            pl.BlockSpec((1, c), zero),
            pl.BlockSpec((4, qw), zero),
        ],
        out_specs=[
            pl.BlockSpec((bn, c), blk),
            pl.BlockSpec((bn, c), blk),
            pl.BlockSpec((bn, c), blk),
            pl.BlockSpec((bn, qw), blk),
        ],
        out_shape=[
            jax.ShapeDtypeStruct((n, c), _F32),
            jax.ShapeDtypeStruct((n, c), _F32),
            jax.ShapeDtypeStruct((n, c), _F32),
            jax.ShapeDtypeStruct((n, qw), _F32),
        ],
    )(x, p4, wq, bq, wk, bk, wv, bv, wp1_16)


def _tc_stats1(gq_wide, q16, msel, msel2, bp1_64, ns, bn):
    n, wide = gq_wide.shape
    qw = wide // ns
    grid = (n // bn,)
    blk = lambda i: (i, 0)
    zero = lambda i: (0, 0)

    def body(gq_r, q16_r, msel_r, msel2_r, b_r, pr3_r, s_r):
        pr3 = (_dot_x(gq_r[...], msel_r[...])
               - _dot_x(q16_r[...], msel2_r[...]) + b_r[...])
        pr3_r[...] = pr3

        @pl.when(pl.program_id(0) == 0)
        def _():
            s_r[...] = jnp.zeros_like(s_r)

        s0 = jnp.sum(pr3, axis=0, keepdims=True)
        s1 = jnp.sum(pr3 * pr3, axis=0, keepdims=True)
        s_r[...] += jnp.concatenate([s0, s1], axis=0)

    return pl.pallas_call(
        body,
        grid=grid,
        in_specs=[
            pl.BlockSpec((bn, wide), blk),
            pl.BlockSpec((bn, qw), blk),
            pl.BlockSpec((wide, 4 * ns), zero),
            pl.BlockSpec((qw, 4 * ns), zero),
            pl.BlockSpec((1, 4 * ns), zero),
        ],
        out_specs=[
            pl.BlockSpec((bn, 4 * ns), blk),
            pl.BlockSpec((2, 4 * ns), zero),
        ],
        out_shape=[
            jax.ShapeDtypeStruct((n, 4 * ns), _F32),
            jax.ShapeDtypeStruct((2, 4 * ns), _F32),
        ],
    )(gq_wide, q16, msel, msel2, bp1_64)


def _tc_stats2(pr3, xq, gk, s_a, pk, tx, bp2_t, gp4, bp4, ns, rtot, bn):
    n, wide = gk.shape
    c = xq.shape[1]
    grid = (n // bn,)
    blk = lambda i: (i, 0)
    zero = lambda i: (0, 0)

    def body(pr3_r, xq_r, gk_r, sa_r, pk_r, tx_r, b2_r, gp_r, bp_r, s_r):
        sc4, sh4 = _bn_scale_shift(_fold(sa_r[...], 4), gp_r[...], bp_r[...], rtot)
        r = jnp.maximum(pr3_r[...] * _tile_lanes(sc4, ns) + _tile_lanes(sh4, ns), 0.0)
        pr = jnp.dot(r, pk_r[...], preferred_element_type=_F32) + b2_r[...]
        w = (gk_r[...].astype(_F32)
             - _dot_x(xq_r[...], tx_r[...]) + pr)

        @pl.when(pl.program_id(0) == 0)
        def _():
            s_r[...] = jnp.zeros_like(s_r)

        s0 = _fold(jnp.sum(w, axis=0, keepdims=True), c)
        s1 = _fold(jnp.sum(w * w, axis=0, keepdims=True), c)
        s_r[...] += jnp.concatenate([s0, s1], axis=0)

    return pl.pallas_call(
        body,
        grid=grid,
        in_specs=[
            pl.BlockSpec((bn, 4 * ns), blk),
            pl.BlockSpec((bn, c), blk),
            pl.BlockSpec((bn, wide), blk),
            pl.BlockSpec((2, 4 * ns), zero),
            pl.BlockSpec(pk.shape, zero),
            pl.BlockSpec(tx.shape, zero),
            pl.BlockSpec((1, wide), zero),
            pl.BlockSpec((1, 4), zero),
            pl.BlockSpec((1, 4), zero),
        ],
        out_specs=pl.BlockSpec((2, c), zero),
        out_shape=jax.ShapeDtypeStruct((2, c), _F32),
    )(pr3, xq, gk, s_a, pk, tx, bp2_t, gp4, bp4)


def _tc_wchain(pr3, xq, gk, s_a, s_b, pk, tx, k1, bp2_t, gp4, bp4,
               g1r, b1r, bw1_t, ns, m, rtot, bn):
    n, wide = gk.shape
    c = xq.shape[1]
    grid = (n // bn,)
    blk = lambda i: (i, 0)
    zero = lambda i: (0, 0)

    def body(pr3_r, xq_r, gk_r, sa_r, sb_r, pk_r, tx_r, k1_r, b2_r,
             gp_r, bp_r, g1_r2, b1_r2, bw1_r, w4a_r, s_r):
        sc4, sh4 = _bn_scale_shift(_fold(sa_r[...], 4), gp_r[...], bp_r[...], rtot)
        r = jnp.maximum(pr3_r[...] * _tile_lanes(sc4, ns) + _tile_lanes(sh4, ns), 0.0)
        pr = jnp.dot(r, pk_r[...], preferred_element_type=_F32) + b2_r[...]
        w = (gk_r[...].astype(_F32)
             - _dot_x(xq_r[...], tx_r[...]) + pr)
        sc32, sh32 = _bn_scale_shift(sb_r[...], g1_r2[...], b1_r2[...], rtot)
        r2 = jnp.maximum(w * _tile_lanes(sc32, ns) + _tile_lanes(sh32, ns), 0.0)
        w4a = jnp.dot(r2, k1_r[...], preferred_element_type=_F32) + bw1_r[...]
        w4a_r[...] = w4a

        @pl.when(pl.program_id(0) == 0)
        def _():
            s_r[...] = jnp.zeros_like(s_r)

        s0 = _fold(jnp.sum(w4a, axis=0, keepdims=True), m)
        s1 = _fold(jnp.sum(w4a * w4a, axis=0, keepdims=True), m)
        s_r[...] += jnp.concatenate([s0, s1], axis=0)

    return pl.pallas_call(
        body,
        grid=grid,
        in_specs=[
            pl.BlockSpec((bn, 4 * ns), blk),
            pl.BlockSpec((bn, c), blk),
            pl.BlockSpec((bn, wide), blk),
            pl.BlockSpec((2, 4 * ns), zero),
            pl.BlockSpec((2, c), zero),
            pl.BlockSpec(pk.shape, zero),
            pl.BlockSpec(tx.shape, zero),
            pl.BlockSpec(k1.shape, zero),
            pl.BlockSpec((1, wide), zero),
            pl.BlockSpec((1, 4), zero),
            pl.BlockSpec((1, 4), zero),
            pl.BlockSpec((1, c), zero),
            pl.BlockSpec((1, c), zero),
            pl.BlockSpec((1, m * ns), zero),
        ],
        out_specs=[
            pl.BlockSpec((bn, m * ns), blk),
            pl.BlockSpec((2, m), zero),
        ],
        out_shape=[
            jax.ShapeDtypeStruct((n, m * ns), _F32),
            jax.ShapeDtypeStruct((2, m), _F32),
        ],
    )(pr3, xq, gk, s_a, s_b, pk, tx, k1, bp2_t, gp4, bp4, g1r, b1r, bw1_t)


def _tc_final(w4a, pr3, gv, s_a, s_c, pk, k2, rw, d64, fold32, bp2_t,
              gp4, bp4, g2r, b2r, bw2_t, ns, m, rtot, bn):
    n, wide = gv.shape
    c = wide // ns
    grid = (n // bn,)
    blk = lambda i: (i, 0)
    zero = lambda i: (0, 0)

    def body(w4a_r, pr3_r, gv_r, sa_r, sc_r, pk_r, k2_r, rw_r, d64_r,
             f32_r, b2_r, gp_r, bp_r, g2_r2, b2_r2, bw2_r, out_r):
        sc4, sh4 = _bn_scale_shift(sc_r[...], g2_r2[...], b2_r2[...], rtot)
        v = jnp.maximum(w4a_r[...] * _tile_lanes(sc4, ns) + _tile_lanes(sh4, ns), 0.0)
        w64 = jnp.dot(v, k2_r[...], preferred_element_type=_F32) + bw2_r[...]
        e = jnp.exp(w64)
        den = jnp.dot(e, d64_r[...], preferred_element_type=_F32)
        sm = e / den
        w512 = jnp.dot(sm, rw_r[...], preferred_element_type=_F32)
        sca, sha = _bn_scale_shift(_fold(sa_r[...], 4), gp_r[...], bp_r[...], rtot)
        r = jnp.maximum(pr3_r[...] * _tile_lanes(sca, ns) + _tile_lanes(sha, ns), 0.0)
        pr = jnp.dot(r, pk_r[...], preferred_element_type=_F32) + b2_r[...]
        t = gv_r[...].astype(_F32) + pr
        out_r[...] = jnp.dot(t * w512, f32_r[...],
                             preferred_element_type=_F32)

    return pl.pallas_call(
        body,
        grid=grid,
        in_specs=[
            pl.BlockSpec((bn, m * ns), blk),
            pl.BlockSpec((bn, 4 * ns), blk),
            pl.BlockSpec((bn, wide), blk),
            pl.BlockSpec((2, 4 * ns), zero),
            pl.BlockSpec((2, m), zero),
            pl.BlockSpec(pk.shape, zero),
            pl.BlockSpec(k2.shape, zero),
            pl.BlockSpec(rw.shape, zero),
            pl.BlockSpec(d64.shape, zero),
            pl.BlockSpec(fold32.shape, zero),
            pl.BlockSpec((1, wide), zero),
            pl.BlockSpec((1, 4), zero),
            pl.BlockSpec((1, 4), zero),
            pl.BlockSpec((1, m), zero),
            pl.BlockSpec((1, m), zero),
            pl.BlockSpec((1, m * ns), zero),
        ],
        out_specs=pl.BlockSpec((bn, c), blk),
        out_shape=jax.ShapeDtypeStruct((n, c), _F32),
    )(w4a, pr3, gv, s_a, s_c, pk, k2, rw, d64, fold32, bp2_t, gp4, bp4,
      g2r, b2r, bw2_t)


def kernel(p, x, edges, Wq, bq, Wk, bk, Wv, bv, Wp1, bp1, gp, bp, Wp2, bp2,
           g1, b1, Ww1, bw1, g2, b2, Ww2, bw2):
    n, c = x.shape
    ns = edges.shape[1]
    m = Ww1.shape[1]
    s = c // m
    rtot = float(n * ns)
    bn = 4000
    assert n % bn == 0

    qw = 16
    eye_ns = jnp.eye(ns, dtype=_F32)
    p4 = jnp.pad(p, ((0, 0), (0, 1)))
    wp1_16 = jnp.pad(Wp1, ((0, 1), (0, qw - 3)))
    wp2_4 = jnp.pad(Wp2, ((0, 1), (0, 0)))
    bp1_64 = jnp.tile(jnp.pad(bp1, (0, 1)), ns)[None]
    bp2_t = jnp.tile(bp2, ns)[None]
    bw1_t = jnp.tile(bw1, ns)[None]
    bw2_t = jnp.tile(bw2, ns)[None]
    gp4 = jnp.pad(gp, (0, 1))[None]
    bp4 = jnp.pad(bp, (0, 1))[None]
    pk = jnp.kron(eye_ns, wp2_4)
    k1 = jnp.kron(eye_ns, Ww1)
    k2 = jnp.kron(eye_ns, Ww2)
    rw = jnp.kron(eye_ns, jnp.kron(jnp.ones((1, s), _F32),
                                   jnp.eye(m, dtype=_F32)))
    msel = jnp.kron(eye_ns, jnp.eye(qw, 4, dtype=_F32))
    msel2 = jnp.kron(jnp.ones((1, ns), _F32),
                     jnp.eye(qw, 4, dtype=_F32))
    tx = jnp.kron(jnp.ones((1, ns), _F32), jnp.eye(c, dtype=_F32))
    d64 = jnp.kron(jnp.ones((ns, ns), _F32), jnp.eye(m, dtype=_F32))
    fold32 = jnp.kron(jnp.ones((ns, 1), _F32), jnp.eye(c, dtype=_F32))

    xq, xk, xv, q16 = _tc_qkv(x, p4, Wq, bq[None], Wk, bk[None], Wv, bv[None],
                              wp1_16, bn)
    gk, gv, gq = _gather_rows(xk, xv, q16, edges.astype(jnp.int32))
    gk_w = gk.reshape(n, ns * c)
    gv_w = gv.reshape(n, ns * c)
    gq_w = gq.reshape(n, ns * qw)

    pr3, s_a = _tc_stats1(gq_w, q16, msel, msel2, bp1_64, ns, bn)
    s_b = _tc_stats2(pr3, xq, gk_w, s_a, pk, tx, bp2_t, gp4, bp4, ns, rtot, bn)
    w4a, s_c = _tc_wchain(pr3, xq, gk_w, s_a, s_b, pk, tx, k1, bp2_t,
                          gp4, bp4, g1[None], b1[None], bw1_t, ns, m, rtot, bn)
    out = _tc_final(w4a, pr3, gv_w, s_a, s_c, pk, k2, rw, d64, fold32, bp2_t,
                    gp4, bp4, g2[None], b2[None], bw2_t, ns, m, rtot, bn)
    return out


# speedup vs baseline: 1.3578x; 1.1997x over previous
import functools

import jax
import jax.numpy as jnp
from jax import lax
from jax.experimental import pallas as pl
from jax.experimental.pallas import tpu as pltpu
from jax.experimental.pallas import tpu_sc as plsc

_EPS = 1e-5
_F32 = jnp.float32


def _dot_x(a, b):
    return jnp.dot(a, b, preferred_element_type=_F32)


def _fold(a, width):
    g = a.shape[-1] // width
    out = a[..., :width]
    for k in range(1, g):
        out = out + a[..., k * width:(k + 1) * width]
    return out


def _tile_lanes(a, reps):
    return jnp.concatenate([a] * reps, axis=-1)


def _bn_scale_shift(s, gamma, beta, count):
    mean = s[0:1, :] / count
    var = s[1:2, :] / count - mean * mean
    inv = lax.rsqrt(var + _EPS)
    scale = gamma * inv
    return scale, beta - mean * scale


def _sc_gather(xk, xv, q16, edges2d):
    n, c = xk.shape
    qw = q16.shape[1]
    ns = edges2d.shape[1]
    b = n * ns
    nw = 32
    per_w = b // nw
    ch = 1000
    assert per_w % ch == 0 and ch % 8 == 0 and per_w * nw == b
    nch = per_w // ch
    mesh = plsc.VectorSubcoreMesh(core_axis_name="c", subcore_axis_name="s")

    out_type = (
        jax.ShapeDtypeStruct((b, c), _F32),
        jax.ShapeDtypeStruct((b, c), _F32),
        jax.ShapeDtypeStruct((b, qw), _F32),
    )
    scratch = [
        pltpu.VMEM((ch,), jnp.int32),
        pltpu.VMEM((ch, c), _F32),
        pltpu.VMEM((ch, c), _F32),
        pltpu.VMEM((ch, qw), _F32),
        pltpu.SemaphoreType.DMA,
        pltpu.SemaphoreType.DMA,
        pltpu.SemaphoreType.DMA,
        pltpu.SemaphoreType.DMA,
        pltpu.SemaphoreType.DMA,
        pltpu.SemaphoreType.DMA,
    ]

    def body(xk_h, xv_h, q16_h, idx_h, gk_h, gv_h, gq_h,
             idx_v, bk, bv, bq, s1, s2, s3, w1, w2, w3):
        wid = lax.axis_index("s") * 2 + lax.axis_index("c")
        base = wid * per_w

        pltpu.async_copy(bk, gk_h.at[pl.ds(base, ch)], w1)
        pltpu.async_copy(bv, gv_h.at[pl.ds(base, ch)], w2)
        pltpu.async_copy(bq, gq_h.at[pl.ds(base, ch)], w3)

        def step(k, carry):
            off = base + k * ch
            pltpu.make_async_copy(bk, gk_h.at[pl.ds(off, ch)], w1).wait()
            pltpu.make_async_copy(bv, gv_h.at[pl.ds(off, ch)], w2).wait()
            pltpu.make_async_copy(bq, gq_h.at[pl.ds(off, ch)], w3).wait()
            pltpu.sync_copy(idx_h.at[pl.ds(off, ch)], idx_v)
            c1 = pltpu.async_copy(xk_h.at[idx_v], bk, s1)
            c2 = pltpu.async_copy(xv_h.at[idx_v], bv, s2)
            c3 = pltpu.async_copy(q16_h.at[idx_v], bq, s3)
            c1.wait()
            c2.wait()
            c3.wait()
            pltpu.async_copy(bk, gk_h.at[pl.ds(off, ch)], w1)
            pltpu.async_copy(bv, gv_h.at[pl.ds(off, ch)], w2)
            pltpu.async_copy(bq, gq_h.at[pl.ds(off, ch)], w3)
            return carry

        lax.fori_loop(0, nch, step, 0)
        last = base + (nch - 1) * ch
        pltpu.make_async_copy(bk, gk_h.at[pl.ds(last, ch)], w1).wait()
        pltpu.make_async_copy(bv, gv_h.at[pl.ds(last, ch)], w2).wait()
        pltpu.make_async_copy(bq, gq_h.at[pl.ds(last, ch)], w3).wait()

    fn = pl.kernel(
        body, out_type=out_type, mesh=mesh, scratch_types=scratch,
        compiler_params=pltpu.CompilerParams(use_tc_tiling_on_sc=False))
    return fn(xk, xv, q16, edges2d.reshape(-1))


_gather_rows = _sc_gather


def _tc_qkv(x, p4, wq, bq, wk, bk, wv, bv, wp1_16, bn):
    n, c = x.shape
    qw = wp1_16.shape[1]
    grid = (n // bn,)
    blk = lambda i: (i, 0)
    zero = lambda i: (0, 0)

    def body(x_r, p4_r, wq_r, bq_r, wk_r, bk_r, wv_r, bv_r, wp_r,
             xq_r, xk_r, xv_r, q_r):
        xb = x_r[...]
        xq_r[...] = jnp.dot(xb, wq_r[...], preferred_element_type=_F32) + bq_r[...]
        xk_r[...] = jnp.dot(xb, wk_r[...], preferred_element_type=_F32) + bk_r[...]
        xv_r[...] = jnp.dot(xb, wv_r[...], preferred_element_type=_F32) + bv_r[...]
        q_r[...] = jnp.dot(p4_r[...], wp_r[...], preferred_element_type=_F32)

    return pl.pallas_call(
        body,
        grid=grid,
        in_specs=[
            pl.BlockSpec((bn, c), blk),
            pl.BlockSpec((bn, 4), blk),
            pl.BlockSpec((c, c), zero),
            pl.BlockSpec((1, c), zero),
            pl.BlockSpec((c, c), zero),
            pl.BlockSpec((1, c), zero),
            pl.BlockSpec((c, c), zero),
            pl.BlockSpec((1, c), zero),
            pl.BlockSpec((4, qw), zero),
        ],
        out_specs=[
            pl.BlockSpec((bn, c), blk),
            pl.BlockSpec((bn, c), blk),
            pl.BlockSpec((bn, c), blk),
            pl.BlockSpec((bn, qw), blk),
        ],
        out_shape=[
            jax.ShapeDtypeStruct((n, c), _F32),
            jax.ShapeDtypeStruct((n, c), _F32),
            jax.ShapeDtypeStruct((n, c), _F32),
            jax.ShapeDtypeStruct((n, qw), _F32),
        ],
    )(x, p4, wq, bq, wk, bk, wv, bv, wp1_16)


def _tc_stats1(gq_wide, q16, msel, msel2, bp1_64, ns, bn):
    n, wide = gq_wide.shape
    qw = wide // ns
    grid = (n // bn,)
    blk = lambda i: (i, 0)
    zero = lambda i: (0, 0)

    def body(gq_r, q16_r, msel_r, msel2_r, b_r, pr3_r, s_r):
        pr3 = (_dot_x(gq_r[...], msel_r[...])
               - _dot_x(q16_r[...], msel2_r[...]) + b_r[...])
        pr3_r[...] = pr3

        @pl.when(pl.program_id(0) == 0)
        def _():
            s_r[...] = jnp.zeros_like(s_r)

        s0 = jnp.sum(pr3, axis=0, keepdims=True)
        s1 = jnp.sum(pr3 * pr3, axis=0, keepdims=True)
        s_r[...] += jnp.concatenate([s0, s1], axis=0)

    return pl.pallas_call(
        body,
        grid=grid,
        in_specs=[
            pl.BlockSpec((bn, wide), blk),
            pl.BlockSpec((bn, qw), blk),
            pl.BlockSpec((wide, 4 * ns), zero),
            pl.BlockSpec((qw, 4 * ns), zero),
            pl.BlockSpec((1, 4 * ns), zero),
        ],
        out_specs=[
            pl.BlockSpec((bn, 4 * ns), blk),
            pl.BlockSpec((2, 4 * ns), zero),
        ],
        out_shape=[
            jax.ShapeDtypeStruct((n, 4 * ns), _F32),
            jax.ShapeDtypeStruct((2, 4 * ns), _F32),
        ],
    )(gq_wide, q16, msel, msel2, bp1_64)


def _tc_stats2(pr3, xq, gk, s_a, pk, tx, bp2_t, gp4, bp4, ns, rtot, bn):
    n, wide = gk.shape
    c = xq.shape[1]
    grid = (n // bn,)
    blk = lambda i: (i, 0)
    zero = lambda i: (0, 0)

    def body(pr3_r, xq_r, gk_r, sa_r, pk_r, tx_r, b2_r, gp_r, bp_r, s_r):
        sc4, sh4 = _bn_scale_shift(_fold(sa_r[...], 4), gp_r[...], bp_r[...], rtot)
        r = jnp.maximum(pr3_r[...] * _tile_lanes(sc4, ns) + _tile_lanes(sh4, ns), 0.0)
        pr = jnp.dot(r, pk_r[...], preferred_element_type=_F32) + b2_r[...]
        w = (gk_r[...].astype(_F32)
             - _dot_x(xq_r[...], tx_r[...]) + pr)

        @pl.when(pl.program_id(0) == 0)
        def _():
            s_r[...] = jnp.zeros_like(s_r)

        s0 = _fold(jnp.sum(w, axis=0, keepdims=True), c)
        s1 = _fold(jnp.sum(w * w, axis=0, keepdims=True), c)
        s_r[...] += jnp.concatenate([s0, s1], axis=0)

    return pl.pallas_call(
        body,
        grid=grid,
        in_specs=[
            pl.BlockSpec((bn, 4 * ns), blk),
            pl.BlockSpec((bn, c), blk),
            pl.BlockSpec((bn, wide), blk),
            pl.BlockSpec((2, 4 * ns), zero),
            pl.BlockSpec(pk.shape, zero),
            pl.BlockSpec(tx.shape, zero),
            pl.BlockSpec((1, wide), zero),
            pl.BlockSpec((1, 4), zero),
            pl.BlockSpec((1, 4), zero),
        ],
        out_specs=pl.BlockSpec((2, c), zero),
        out_shape=jax.ShapeDtypeStruct((2, c), _F32),
    )(pr3, xq, gk, s_a, pk, tx, bp2_t, gp4, bp4)


def _tc_wchain(pr3, xq, gk, s_a, s_b, pk, tx, k1, bp2_t, gp4, bp4,
               g1r, b1r, bw1_t, ns, m, rtot, bn):
    n, wide = gk.shape
    c = xq.shape[1]
    grid = (n // bn,)
    blk = lambda i: (i, 0)
    zero = lambda i: (0, 0)

    def body(pr3_r, xq_r, gk_r, sa_r, sb_r, pk_r, tx_r, k1_r, b2_r,
             gp_r, bp_r, g1_r2, b1_r2, bw1_r, w4a_r, s_r):
        sc4, sh4 = _bn_scale_shift(_fold(sa_r[...], 4), gp_r[...], bp_r[...], rtot)
        r = jnp.maximum(pr3_r[...] * _tile_lanes(sc4, ns) + _tile_lanes(sh4, ns), 0.0)
        pr = jnp.dot(r, pk_r[...], preferred_element_type=_F32) + b2_r[...]
        w = (gk_r[...].astype(_F32)
             - _dot_x(xq_r[...], tx_r[...]) + pr)
        sc32, sh32 = _bn_scale_shift(sb_r[...], g1_r2[...], b1_r2[...], rtot)
        r2 = jnp.maximum(w * _tile_lanes(sc32, ns) + _tile_lanes(sh32, ns), 0.0)
        w4a = jnp.dot(r2, k1_r[...], preferred_element_type=_F32) + bw1_r[...]
        w4a_r[...] = w4a

        @pl.when(pl.program_id(0) == 0)
        def _():
            s_r[...] = jnp.zeros_like(s_r)

        s0 = _fold(jnp.sum(w4a, axis=0, keepdims=True), m)
        s1 = _fold(jnp.sum(w4a * w4a, axis=0, keepdims=True), m)
        s_r[...] += jnp.concatenate([s0, s1], axis=0)

    return pl.pallas_call(
        body,
        grid=grid,
        in_specs=[
            pl.BlockSpec((bn, 4 * ns), blk),
            pl.BlockSpec((bn, c), blk),
            pl.BlockSpec((bn, wide), blk),
            pl.BlockSpec((2, 4 * ns), zero),
            pl.BlockSpec((2, c), zero),
            pl.BlockSpec(pk.shape, zero),
            pl.BlockSpec(tx.shape, zero),
            pl.BlockSpec(k1.shape, zero),
            pl.BlockSpec((1, wide), zero),
            pl.BlockSpec((1, 4), zero),
            pl.BlockSpec((1, 4), zero),
            pl.BlockSpec((1, c), zero),
            pl.BlockSpec((1, c), zero),
            pl.BlockSpec((1, m * ns), zero),
        ],
        out_specs=[
            pl.BlockSpec((bn, m * ns), blk),
            pl.BlockSpec((2, m), zero),
        ],
        out_shape=[
            jax.ShapeDtypeStruct((n, m * ns), _F32),
            jax.ShapeDtypeStruct((2, m), _F32),
        ],
    )(pr3, xq, gk, s_a, s_b, pk, tx, k1, bp2_t, gp4, bp4, g1r, b1r, bw1_t)


def _tc_final(w4a, pr3, gv, s_a, s_c, pk, k2, rw, d64, fold32, bp2_t,
              gp4, bp4, g2r, b2r, bw2_t, ns, m, rtot, bn):
    n, wide = gv.shape
    c = wide // ns
    grid = (n // bn,)
    blk = lambda i: (i, 0)
    zero = lambda i: (0, 0)

    def body(w4a_r, pr3_r, gv_r, sa_r, sc_r, pk_r, k2_r, rw_r, d64_r,
             f32_r, b2_r, gp_r, bp_r, g2_r2, b2_r2, bw2_r, out_r):
        sc4, sh4 = _bn_scale_shift(sc_r[...], g2_r2[...], b2_r2[...], rtot)
        v = jnp.maximum(w4a_r[...] * _tile_lanes(sc4, ns) + _tile_lanes(sh4, ns), 0.0)
        w64 = jnp.dot(v, k2_r[...], preferred_element_type=_F32) + bw2_r[...]
        e = jnp.exp(w64)
        den = jnp.dot(e, d64_r[...], preferred_element_type=_F32)
        sm = e / den
        w512 = jnp.dot(sm, rw_r[...], preferred_element_type=_F32)
        sca, sha = _bn_scale_shift(_fold(sa_r[...], 4), gp_r[...], bp_r[...], rtot)
        r = jnp.maximum(pr3_r[...] * _tile_lanes(sca, ns) + _tile_lanes(sha, ns), 0.0)
        pr = jnp.dot(r, pk_r[...], preferred_element_type=_F32) + b2_r[...]
        t = gv_r[...].astype(_F32) + pr
        out_r[...] = jnp.dot(t * w512, f32_r[...],
                             preferred_element_type=_F32)

    return pl.pallas_call(
        body,
        grid=grid,
        in_specs=[
            pl.BlockSpec((bn, m * ns), blk),
            pl.BlockSpec((bn, 4 * ns), blk),
            pl.BlockSpec((bn, wide), blk),
            pl.BlockSpec((2, 4 * ns), zero),
            pl.BlockSpec((2, m), zero),
            pl.BlockSpec(pk.shape, zero),
            pl.BlockSpec(k2.shape, zero),
            pl.BlockSpec(rw.shape, zero),
            pl.BlockSpec(d64.shape, zero),
            pl.BlockSpec(fold32.shape, zero),
            pl.BlockSpec((1, wide), zero),
            pl.BlockSpec((1, 4), zero),
            pl.BlockSpec((1, 4), zero),
            pl.BlockSpec((1, m), zero),
            pl.BlockSpec((1, m), zero),
            pl.BlockSpec((1, m * ns), zero),
        ],
        out_specs=pl.BlockSpec((bn, c), blk),
        out_shape=jax.ShapeDtypeStruct((n, c), _F32),
    )(w4a, pr3, gv, s_a, s_c, pk, k2, rw, d64, fold32, bp2_t, gp4, bp4,
      g2r, b2r, bw2_t)


def kernel(p, x, edges, Wq, bq, Wk, bk, Wv, bv, Wp1, bp1, gp, bp, Wp2, bp2,
           g1, b1, Ww1, bw1, g2, b2, Ww2, bw2):
    n, c = x.shape
    ns = edges.shape[1]
    m = Ww1.shape[1]
    s = c // m
    rtot = float(n * ns)
    bn = 4000
    assert n % bn == 0

    qw = 16
    eye_ns = jnp.eye(ns, dtype=_F32)
    p4 = jnp.pad(p, ((0, 0), (0, 1)))
    wp1_16 = jnp.pad(Wp1, ((0, 1), (0, qw - 3)))
    wp2_4 = jnp.pad(Wp2, ((0, 1), (0, 0)))
    bp1_64 = jnp.tile(jnp.pad(bp1, (0, 1)), ns)[None]
    bp2_t = jnp.tile(bp2, ns)[None]
    bw1_t = jnp.tile(bw1, ns)[None]
    bw2_t = jnp.tile(bw2, ns)[None]
    gp4 = jnp.pad(gp, (0, 1))[None]
    bp4 = jnp.pad(bp, (0, 1))[None]
    pk = jnp.kron(eye_ns, wp2_4)
    k1 = jnp.kron(eye_ns, Ww1)
    k2 = jnp.kron(eye_ns, Ww2)
    rw = jnp.kron(eye_ns, jnp.kron(jnp.ones((1, s), _F32),
                                   jnp.eye(m, dtype=_F32)))
    msel = jnp.kron(eye_ns, jnp.eye(qw, 4, dtype=_F32))
    msel2 = jnp.kron(jnp.ones((1, ns), _F32),
                     jnp.eye(qw, 4, dtype=_F32))
    tx = jnp.kron(jnp.ones((1, ns), _F32), jnp.eye(c, dtype=_F32))
    d64 = jnp.kron(jnp.ones((ns, ns), _F32), jnp.eye(m, dtype=_F32))
    fold32 = jnp.kron(jnp.ones((ns, 1), _F32), jnp.eye(c, dtype=_F32))

    xq, xk, xv, q16 = _tc_qkv(x, p4, Wq, bq[None], Wk, bk[None], Wv, bv[None],
                              wp1_16, bn)
    gk, gv, gq = _gather_rows(xk, xv, q16, edges.astype(jnp.int32))
    gk_w = gk.reshape(n, ns * c)
    gv_w = gv.reshape(n, ns * c)
    gq_w = gq.reshape(n, ns * qw)

    pr3, s_a = _tc_stats1(gq_w, q16, msel, msel2, bp1_64, ns, bn)
    s_b = _tc_stats2(pr3, xq, gk_w, s_a, pk, tx, bp2_t, gp4, bp4, ns, rtot, bn)
    w4a, s_c = _tc_wchain(pr3, xq, gk_w, s_a, s_b, pk, tx, k1, bp2_t,
                          gp4, bp4, g1[None], b1[None], bw1_t, ns, m, rtot, bn)
    out = _tc_final(w4a, pr3, gv_w, s_a, s_c, pk, k2, rw, d64, fold32, bp2_t,
                    gp4, bp4, g2[None], b2[None], bw2_t, ns, m, rtot, bn)
    return out
